# Initial kernel scaffold; baseline (speedup 1.0000x reference)
import jax
import jax.numpy as jnp
from jax.experimental import pallas as pl


def kernel(x, bee_positions, bee_fitness, best_position, best_fitness, exploration_radius, exploitation_radius):
    raise NotImplementedError("write your pallas kernel here")


# speedup vs baseline: 5.9913x; 5.9913x over previous
import jax
import jax.numpy as jnp
from jax.experimental import pallas as pl

_NUM_BEES = 40
_BLK = 512


def _fitness_kernel(x_ref, p_ref, psq_ref, out_ref):
    i = pl.program_id(0)
    x = x_ref[...]
    dot = jax.lax.dot_general(
        x, p_ref[...],
        dimension_numbers=(((1,), (1,)), ((), ())),
        preferred_element_type=jnp.float32,
        precision=jax.lax.Precision.HIGHEST,
    )
    x_sq = jnp.sum(x * x, axis=1, keepdims=True)
    sq = jnp.maximum(x_sq - 2.0 * dot + psq_ref[...], 0.0)
    partial = jnp.sum(jnp.sqrt(sq), axis=0, keepdims=True)

    @pl.when(i == 0)
    def _():
        out_ref[...] = jnp.zeros_like(out_ref)

    out_ref[...] += partial


def _add_kernel(sums_ref, p_ref, bestpos_ref, bestfit_ref, x_ref, out_ref,
                *, inv_n):
    sums = sums_ref[...]
    max_sum = jnp.max(sums)
    iota = jax.lax.broadcasted_iota(jnp.int32, (1, _NUM_BEES), 1)
    idx = jnp.min(jnp.where(sums == max_sum, iota, _NUM_BEES))
    onehot = jax.lax.broadcasted_iota(jnp.int32, (_NUM_BEES, 1), 0) == idx
    chosen = jnp.sum(jnp.where(onehot, p_ref[...], 0.0), axis=0,
                     keepdims=True)
    better = max_sum * inv_n > bestfit_ref[0, 0]
    add = jnp.where(better, chosen, bestpos_ref[...])
    out_ref[...] = x_ref[...] + add


def kernel(x, bee_positions, bee_fitness, best_position, best_fitness,
           exploration_radius, exploitation_radius):
    B, S, H = x.shape
    n_rows = B * S
    xr = x.reshape(n_rows, H)
    n_blk = n_rows // _BLK
    psq = jnp.sum(bee_positions * bee_positions, axis=1, keepdims=True).T

    sums = pl.pallas_call(
        _fitness_kernel,
        grid=(n_blk,),
        in_specs=[
            pl.BlockSpec((_BLK, H), lambda i: (i, 0)),
            pl.BlockSpec((_NUM_BEES, H), lambda i: (0, 0)),
            pl.BlockSpec((1, _NUM_BEES), lambda i: (0, 0)),
        ],
        out_specs=pl.BlockSpec((1, _NUM_BEES), lambda i: (0, 0)),
        out_shape=jax.ShapeDtypeStruct((1, _NUM_BEES), jnp.float32),
    )(xr, bee_positions, psq)

    out = pl.pallas_call(
        lambda *refs: _add_kernel(*refs, inv_n=1.0 / n_rows),
        grid=(n_blk,),
        in_specs=[
            pl.BlockSpec((1, _NUM_BEES), lambda i: (0, 0)),
            pl.BlockSpec((_NUM_BEES, H), lambda i: (0, 0)),
            pl.BlockSpec((1, H), lambda i: (0, 0)),
            pl.BlockSpec((1, 1), lambda i: (0, 0)),
            pl.BlockSpec((_BLK, H), lambda i: (i, 0)),
        ],
        out_specs=pl.BlockSpec((_BLK, H), lambda i: (i, 0)),
        out_shape=jax.ShapeDtypeStruct((n_rows, H), jnp.float32),
    )(sums, bee_positions, best_position.reshape(1, H),
      best_fitness.reshape(1, 1), xr)

    return out.reshape(B, S, H)


# speedup vs baseline: 7.2382x; 1.2081x over previous
import jax
import jax.numpy as jnp
from jax.experimental import pallas as pl
from jax.experimental.pallas import tpu as pltpu

_NUM_BEES = 40
_BLK = 512


def _bee_kernel(p_ref, psq_ref, bestpos_ref, bestfit_ref, x_ref, out_ref,
                xs_ref, acc_ref, *, inv_n, n_blk):
    phase = pl.program_id(0)
    i = pl.program_id(1)

    @pl.when(phase == 0)
    def _():
        x = x_ref[...]
        xs_ref[pl.ds(i * _BLK, _BLK), :] = x
        dot = jax.lax.dot_general(
            x, p_ref[...],
            dimension_numbers=(((1,), (1,)), ((), ())),
            preferred_element_type=jnp.float32,
            precision=jax.lax.Precision.HIGHEST,
        )
        x_sq = jnp.sum(x * x, axis=1, keepdims=True)
        sq = jnp.maximum(x_sq - 2.0 * dot + psq_ref[...], 0.0)
        partial = jnp.sum(jnp.sqrt(sq), axis=0, keepdims=True)

        @pl.when(i == 0)
        def _():
            acc_ref[...] = jnp.zeros_like(acc_ref)

        acc_ref[...] += partial

    @pl.when(phase == 1)
    def _():
        sums = acc_ref[...]
        max_sum = jnp.max(sums)
        iota = jax.lax.broadcasted_iota(jnp.int32, (1, _NUM_BEES), 1)
        idx = jnp.min(jnp.where(sums == max_sum, iota, _NUM_BEES))
        onehot = jax.lax.broadcasted_iota(jnp.int32, (_NUM_BEES, 1), 0) == idx
        chosen = jnp.sum(jnp.where(onehot, p_ref[...], 0.0), axis=0,
                         keepdims=True)
        better = max_sum * inv_n > bestfit_ref[0, 0]
        add = jnp.where(better, chosen, bestpos_ref[...])
        out_ref[...] = xs_ref[pl.ds(i * _BLK, _BLK), :] + add


def kernel(x, bee_positions, bee_fitness, best_position, best_fitness,
           exploration_radius, exploitation_radius):
    B, S, H = x.shape
    n_rows = B * S
    xr = x.reshape(n_rows, H)
    n_blk = n_rows // _BLK
    psq = jnp.sum(bee_positions * bee_positions, axis=1, keepdims=True).T

    import functools
    body = functools.partial(_bee_kernel, inv_n=1.0 / n_rows, n_blk=n_blk)

    out = pl.pallas_call(
        body,
        grid=(2, n_blk),
        in_specs=[
            pl.BlockSpec((_NUM_BEES, H), lambda p, i: (0, 0)),
            pl.BlockSpec((1, _NUM_BEES), lambda p, i: (0, 0)),
            pl.BlockSpec((1, H), lambda p, i: (0, 0)),
            pl.BlockSpec((1, 1), lambda p, i: (0, 0)),
            pl.BlockSpec((_BLK, H), lambda p, i: (i * (1 - p), 0)),
        ],
        out_specs=pl.BlockSpec((_BLK, H), lambda p, i: (i * p, 0)),
        out_shape=jax.ShapeDtypeStruct((n_rows, H), jnp.float32),
        scratch_shapes=[
            pltpu.VMEM((n_rows, H), jnp.float32),
            pltpu.VMEM((1, _NUM_BEES), jnp.float32),
        ],
    )(bee_positions, psq, best_position.reshape(1, H),
      best_fitness.reshape(1, 1), xr)

    return out.reshape(B, S, H)


# speedup vs baseline: 9.4164x; 1.3009x over previous
import jax
import jax.numpy as jnp
from jax.experimental import pallas as pl
from jax.experimental.pallas import tpu as pltpu

_NUM_BEES = 40
_BLK = 512


def _bee_kernel(p_ref, psq_ref, bestpos_ref, bestfit_ref, x_ref, out_ref,
                xs_ref, acc_ref, add_ref, *, inv_n, n_blk):
    phase = pl.program_id(0)
    i = pl.program_id(1)

    @pl.when(phase == 0)
    def _():
        x = x_ref[...]
        xs_ref[pl.ds(i * _BLK, _BLK), :] = x
        x_hi = x.astype(jnp.bfloat16)
        x_lo = (x - x_hi.astype(jnp.float32)).astype(jnp.bfloat16)
        p = p_ref[...]
        p_hi = p.astype(jnp.bfloat16)
        p_lo = (p - p_hi.astype(jnp.float32)).astype(jnp.bfloat16)
        dn = (((1,), (1,)), ((), ()))

        def bdot(a, b):
            return jax.lax.dot_general(
                a, b, dimension_numbers=dn,
                preferred_element_type=jnp.float32)

        dot = bdot(x_hi, p_hi) + bdot(x_hi, p_lo) + bdot(x_lo, p_hi)
        x_sq = jnp.sum(x * x, axis=1, keepdims=True)
        sq = jnp.maximum(x_sq - 2.0 * dot + psq_ref[...], 0.0)
        partial = jnp.sum(jnp.sqrt(sq), axis=0, keepdims=True)

        @pl.when(i == 0)
        def _():
            acc_ref[...] = jnp.zeros_like(acc_ref)

        acc_ref[...] += partial

    @pl.when((phase == 1) & (i == 0))
    def _():
        sums = acc_ref[...]
        max_sum = jnp.max(sums)
        iota = jax.lax.broadcasted_iota(jnp.int32, (1, _NUM_BEES), 1)
        idx = jnp.min(jnp.where(sums == max_sum, iota, _NUM_BEES))
        onehot = jax.lax.broadcasted_iota(jnp.int32, (_NUM_BEES, 1), 0) == idx
        chosen = jnp.sum(jnp.where(onehot, p_ref[...], 0.0), axis=0,
                         keepdims=True)
        better = max_sum * inv_n > bestfit_ref[0, 0]
        add_ref[...] = jnp.where(better, chosen, bestpos_ref[...])

    @pl.when(phase == 1)
    def _():
        out_ref[...] = xs_ref[pl.ds(i * _BLK, _BLK), :] + add_ref[...]


def kernel(x, bee_positions, bee_fitness, best_position, best_fitness,
           exploration_radius, exploitation_radius):
    B, S, H = x.shape
    n_rows = B * S
    xr = x.reshape(n_rows, H)
    n_blk = n_rows // _BLK
    psq = jnp.sum(bee_positions * bee_positions, axis=1, keepdims=True).T

    import functools
    body = functools.partial(_bee_kernel, inv_n=1.0 / n_rows, n_blk=n_blk)

    out = pl.pallas_call(
        body,
        grid=(2, n_blk),
        in_specs=[
            pl.BlockSpec((_NUM_BEES, H), lambda p, i: (0, 0)),
            pl.BlockSpec((1, _NUM_BEES), lambda p, i: (0, 0)),
            pl.BlockSpec((1, H), lambda p, i: (0, 0)),
            pl.BlockSpec((1, 1), lambda p, i: (0, 0)),
            pl.BlockSpec((_BLK, H), lambda p, i: (i * (1 - p), 0)),
        ],
        out_specs=pl.BlockSpec((_BLK, H), lambda p, i: (i * p, 0)),
        out_shape=jax.ShapeDtypeStruct((n_rows, H), jnp.float32),
        scratch_shapes=[
            pltpu.VMEM((n_rows, H), jnp.float32),
            pltpu.VMEM((1, _NUM_BEES), jnp.float32),
            pltpu.VMEM((1, H), jnp.float32),
        ],
    )(bee_positions, psq, best_position.reshape(1, H),
      best_fitness.reshape(1, 1), xr)

    return out.reshape(B, S, H)
